# baseline (speedup 1.0000x reference)
import functools

import jax
import jax.numpy as jnp
from jax import lax
from jax.experimental import pallas as pl
from jax.experimental.pallas import tpu as pltpu
from jax.experimental.pallas import tpu_sc as plsc

NC, NS = 2, 16
NW = NC * NS
V = 48
D = 64
L = 16
CHUNK = 640
NB = 2


@functools.lru_cache(maxsize=None)
def _make_sc_gather(B: int):
    assert B % (NW * CHUNK * NB) == 0
    b_per_w = B // NW
    n_chunks = b_per_w // CHUNK
    mesh = plsc.VectorSubcoreMesh(core_axis_name="c", subcore_axis_name="s")

    @functools.partial(
        pl.kernel,
        mesh=mesh,
        out_type=jax.ShapeDtypeStruct((B, D), jnp.float32),
        scratch_types=[
            pltpu.VMEM((V, D), jnp.float32),
            pltpu.VMEM((NB, CHUNK), jnp.int32),
            pltpu.VMEM((NB, CHUNK, D), jnp.float32),
            pltpu.SemaphoreType.DMA((NB,)),
            pltpu.SemaphoreType.DMA((NB,)),
        ],
        compiler_params=pltpu.CompilerParams(use_tc_tiling_on_sc=False,
                                             needs_layout_passes=False),
    )
    def k(table_hbm, idx_hbm, out_hbm, table_v, idx_v, rows_v, idx_sem,
          out_sem):
        wid = lax.axis_index("s") * NC + lax.axis_index("c")
        row0 = wid * b_per_w

        def idx_copy(g, s):
            return pltpu.make_async_copy(
                idx_hbm.at[pl.ds(row0 + g * CHUNK, CHUNK)],
                idx_v.at[s], idx_sem.at[s])

        def out_copy(g, s):
            return pltpu.make_async_copy(
                rows_v.at[s],
                out_hbm.at[pl.ds(row0 + g * CHUNK, CHUNK)], out_sem.at[s])

        pltpu.sync_copy(table_hbm, table_v)
        for s in range(NB):
            idx_copy(s, s).start()

        def expand(s):
            dst = rows_v.at[s]
            src = idx_v.at[s]

            def body(i, carry):
                idx16 = src[pl.ds(i * L, L)]
                row16 = lax.iota(jnp.int32, L) + i * L
                for j in range(D):
                    col = jnp.full((L,), j, jnp.int32)
                    vals = plsc.load_gather(table_v, [idx16, col])
                    plsc.store_scatter(dst, [row16, col], vals)
                return carry

            lax.fori_loop(0, CHUNK // L, body, 0)

        def outer(i, carry):
            g0 = i * NB
            for s in range(NB):
                g = g0 + s
                idx_copy(g, s).wait()

                @pl.when(g >= NB)
                def _():
                    out_copy(g - NB, s).wait()

                expand(s)

                @pl.when(g + NB < n_chunks)
                def _():
                    idx_copy(g + NB, s).start()

                out_copy(g, s).start()
            return carry

        lax.fori_loop(0, n_chunks // NB, outer, 0)
        for s in range(NB):
            out_copy(n_chunks - NB + s, s).wait()

    return k


def kernel(timedelta, table):
    Bt, H = timedelta.shape
    B = Bt * H
    idx = timedelta.reshape(B).astype(jnp.int32)
    out = _make_sc_gather(B)(table, idx)
    return out.reshape(Bt, H, D)


# speedup vs baseline: 4.0854x; 4.0854x over previous
import functools

import jax
import jax.numpy as jnp
from jax import lax
from jax.experimental import pallas as pl
from jax.experimental.pallas import tpu as pltpu
from jax.experimental.pallas import tpu_sc as plsc

NC, NS = 2, 16
NW = NC * NS
V = 48
D = 64
CHUNK = 640
IDX_SUB = CHUNK // 128
NB = 2


@functools.lru_cache(maxsize=None)
def _make_sc_gather(B: int):
    assert B % (NW * CHUNK * NB) == 0
    b_per_w = B // NW
    n_chunks = b_per_w // CHUNK
    idx_rows_per_w = b_per_w // 128
    mesh = plsc.VectorSubcoreMesh(core_axis_name="c", subcore_axis_name="s")

    @functools.partial(
        pl.kernel,
        mesh=mesh,
        out_type=jax.ShapeDtypeStruct((B, D), jnp.float32),
        scratch_types=[
            pltpu.VMEM_SHARED((V, D), jnp.float32),
            pltpu.VMEM((NB, IDX_SUB, 128), jnp.int32),
            pltpu.VMEM((NB, CHUNK, D), jnp.float32),
            pltpu.SemaphoreType.DMA((NB,)),
            pltpu.SemaphoreType.DMA((NB,)),
            pltpu.SemaphoreType.DMA((NB,)),
        ],
        compiler_params=pltpu.CompilerParams(use_tc_tiling_on_sc=False,
                                             needs_layout_passes=False),
    )
    def k(table_hbm, idx_hbm, out_hbm, table_v, idx_v, rows_v, idx_sem,
          gat_sem, out_sem):
        wid = lax.axis_index("s") * NC + lax.axis_index("c")
        irow0 = wid * idx_rows_per_w
        orow0 = wid * b_per_w

        def idx_copy(g, s):
            return pltpu.make_async_copy(
                idx_hbm.at[pl.ds(irow0 + g * IDX_SUB, IDX_SUB)],
                idx_v.at[s], idx_sem.at[s])

        def gat_copy(s, j):
            return pltpu.make_async_copy(
                table_v.at[idx_v.at[s].at[j]],
                rows_v.at[s].at[pl.ds(j * 128, 128)], gat_sem.at[s])

        def out_copy(g, s):
            return pltpu.make_async_copy(
                rows_v.at[s],
                out_hbm.at[pl.ds(orow0 + g * CHUNK, CHUNK)], out_sem.at[s])

        @pl.when(lax.axis_index("s") == 0)
        def _():
            pltpu.sync_copy(table_hbm, table_v)

        plsc.subcore_barrier()
        for s in range(NB):
            idx_copy(s, s).start()

        def outer(i, carry):
            g0 = i * NB
            for s in range(NB):
                g = g0 + s
                idx_copy(g, s).wait()

                @pl.when(g >= NB)
                def _():
                    out_copy(g - NB, s).wait()

                for j in range(IDX_SUB):
                    gat_copy(s, j).start()
                for j in range(IDX_SUB):
                    gat_copy(s, j).wait()

                @pl.when(g + NB < n_chunks)
                def _():
                    idx_copy(g + NB, s).start()

                out_copy(g, s).start()
            return carry

        lax.fori_loop(0, n_chunks // NB, outer, 0)
        for s in range(NB):
            out_copy(n_chunks - NB + s, s).wait()

    return k


def kernel(timedelta, table):
    Bt, H = timedelta.shape
    B = Bt * H
    idx = timedelta.reshape(B // 128, 128).astype(jnp.int32)
    out = _make_sc_gather(B)(table, idx)
    return out.reshape(Bt, H, D)
